# baseline (speedup 1.0000x reference)
import functools

import jax
import jax.numpy as jnp
from jax import lax
from jax.experimental import pallas as pl
from jax.experimental.pallas import tpu as pltpu
from jax.experimental.pallas import tpu_sc as plsc

B, N, C = 8, 20000, 91
DM, DG = 256, 4
K = 300
KPAD = 320
NCHUNK = 1280
NSTEPS = 16
ROWS = NCHUNK // 128
TOTROWS = NSTEPS * ROWS
NW = 32
RPW = B * KPAD // NW


def _topk_body(cls_ref, idx_ref, scores_ref):
    i = pl.program_id(0)
    blk = jnp.max(cls_ref[...], axis=2)
    pos = i * NCHUNK + lax.broadcasted_iota(jnp.int32, (B, NCHUNK), 1)
    blk = jnp.where(pos >= N, -jnp.inf, blk)
    scores_ref[:, pl.ds(i * ROWS, ROWS), :] = blk.reshape(B, ROWS, 128)

    @pl.when(i == NSTEPS - 1)
    def _extract():
        lane128 = lax.broadcasted_iota(jnp.int32, (1, 128), 1)
        iota160 = lax.broadcasted_iota(jnp.int32, (1, TOTROWS), 1)
        lanek = lax.broadcasted_iota(jnp.int32, (1, KPAD), 1)

        cm0 = tuple(
            jnp.max(scores_ref[b], axis=1).reshape(1, TOTROWS) for b in range(B)
        )
        idx0 = tuple(jnp.zeros((1, KPAD), jnp.int32) for _ in range(B))

        def body(j, carry):
            cm, idxs = carry
            new_cm = []
            new_idxs = []
            for b in range(B):
                cmax = jnp.max(cm[b])
                c_b = jnp.min(jnp.where(cm[b] == cmax, iota160, jnp.int32(2**30)))
                row = scores_ref[b, pl.ds(c_b, 1), :]
                rmax = jnp.max(row)
                l_b = jnp.min(jnp.where(row == rmax, lane128, jnp.int32(2**30)))
                row_m = jnp.where(lane128 == l_b, -jnp.inf, row)
                scores_ref[b, pl.ds(c_b, 1), :] = row_m
                new_cm.append(jnp.where(iota160 == c_b, jnp.max(row_m), cm[b]))
                tok = c_b * 128 + l_b + b * N
                new_idxs.append(jnp.where(lanek == j, tok, idxs[b]))
            return tuple(new_cm), tuple(new_idxs)

        _, idxs = lax.fori_loop(0, K, body, (cm0, idx0))
        idx_ref[...] = jnp.concatenate(idxs, axis=0)


def _tc_topk(class_logits):
    return pl.pallas_call(
        _topk_body,
        grid=(NSTEPS,),
        in_specs=[pl.BlockSpec((B, NCHUNK, C), lambda i: (0, i, 0))],
        out_specs=pl.BlockSpec((B, KPAD), lambda i: (0, 0)),
        out_shape=jax.ShapeDtypeStruct((B, KPAD), jnp.int32),
        scratch_shapes=[pltpu.VMEM((B, TOTROWS, 128), jnp.float32)],
    )(class_logits)


def _sc_gather_body(mem_hbm, idx_hbm, mem_out, idx_v, mem_v, sem):
    wid = lax.axis_index("s") * 2 + lax.axis_index("c")
    base = wid * RPW
    pltpu.sync_copy(idx_hbm.at[pl.ds(base, RPW)], idx_v)
    pltpu.async_copy(mem_hbm.at[idx_v], mem_v, sem).wait()
    pltpu.sync_copy(mem_v, mem_out.at[pl.ds(base, RPW)])


def _sc_gather(mem2, idx_flat):
    mesh = plsc.VectorSubcoreMesh(core_axis_name="c", subcore_axis_name="s")
    run = functools.partial(
        pl.kernel,
        mesh=mesh,
        out_type=jax.ShapeDtypeStruct((B * KPAD, DM), jnp.float32),
        scratch_types=[
            pltpu.VMEM((RPW,), jnp.int32),
            pltpu.VMEM((RPW, DM), jnp.float32),
            pltpu.SemaphoreType.DMA,
        ],
    )(_sc_gather_body)
    return run(mem2, idx_flat)


def _tc_gather_body(idx_ref, cls_ref, geo_ref, clso_ref, geoo_ref):
    b = pl.program_id(0)

    def body(j, carry):
        t = idx_ref[b, j] - b * N
        clso_ref[0, pl.ds(j, 1), :] = cls_ref[0, pl.ds(t, 1), :]
        geoo_ref[0, pl.ds(j, 1), :] = geo_ref[0, pl.ds(t, 1), :]
        return carry

    lax.fori_loop(0, K, body, 0)


def _tc_gather(class_logits, geometry_logits, idx):
    grid_spec = pltpu.PrefetchScalarGridSpec(
        num_scalar_prefetch=1,
        grid=(B,),
        in_specs=[
            pl.BlockSpec((1, N, C), lambda b, idx_ref: (b, 0, 0)),
            pl.BlockSpec((1, N, DG), lambda b, idx_ref: (b, 0, 0)),
        ],
        out_specs=[
            pl.BlockSpec((1, KPAD, C), lambda b, idx_ref: (b, 0, 0)),
            pl.BlockSpec((1, KPAD, DG), lambda b, idx_ref: (b, 0, 0)),
        ],
    )
    return pl.pallas_call(
        _tc_gather_body,
        grid_spec=grid_spec,
        out_shape=[
            jax.ShapeDtypeStruct((B, KPAD, C), jnp.float32),
            jax.ShapeDtypeStruct((B, KPAD, DG), jnp.float32),
        ],
    )(idx, class_logits, geometry_logits)


def kernel(memory, class_logits, geometry_logits):
    idx = _tc_topk(class_logits)
    mem2 = memory.reshape(B * N, DM)
    mem_g = _sc_gather(mem2, idx.reshape(-1))
    cls_g, geo_g = _tc_gather(class_logits, geometry_logits, idx)
    topk_memory = mem_g.reshape(B, KPAD, DM)[:, :K]
    topk_logits = cls_g[:, :K]
    topk_coords = geo_g[:, :K]
    return (topk_memory, topk_logits, topk_coords)
